# baseline (speedup 1.0000x reference)
import functools

import jax
import jax.numpy as jnp
from jax import lax
from jax.experimental import pallas as pl
from jax.experimental.pallas import tpu as pltpu
from jax.experimental.pallas import tpu_sc as plsc

E = 1_600_000
N = 100_000
C_IN = 32
C_OUT = 32
KVOL = 27

_info = plsc.get_sparse_core_info()
NC = _info.num_cores
NS = _info.num_subcores
L = _info.num_lanes
NW = NC * NS

EH = E // 2
GCH = 1000
EPW = EH // NW
GITERS = EPW // GCH


def _gather_sc(feats, edge_index, e0):
  mesh = plsc.VectorSubcoreMesh(core_axis_name="c", subcore_axis_name="s")

  @functools.partial(
      pl.kernel,
      mesh=mesh,
      compiler_params=pltpu.CompilerParams(use_tc_tiling_on_sc=False),
      out_type=jax.ShapeDtypeStruct((EH, C_IN), jnp.float32),
      scratch_types=[
          pltpu.VMEM((EPW,), jnp.int32),
          pltpu.VMEM((2, GCH, C_IN), jnp.float32),
          pltpu.SemaphoreType.DMA,
          pltpu.SemaphoreType.DMA((2,)),
      ],
  )
  def k(feats_hbm, ei_hbm, out_hbm, idx_v, rows_v, sem_g, sem_w):
    wid = lax.axis_index("s") * NC + lax.axis_index("c")
    base = wid * EPW
    pltpu.sync_copy(ei_hbm.at[0, pl.ds(e0 + base, EPW)], idx_v)

    def body(i, carry):
      p = lax.rem(i, 2)
      off = base + i * GCH

      @pl.when(i >= 2)
      def _():
        off2 = base + (i - 2) * GCH
        pltpu.make_async_copy(rows_v.at[p], out_hbm.at[pl.ds(off2, GCH)],
                              sem_w.at[p]).wait()

      pltpu.async_copy(feats_hbm.at[idx_v.at[pl.ds(i * GCH, GCH)]],
                       rows_v.at[p], sem_g).wait()
      pltpu.async_copy(rows_v.at[p], out_hbm.at[pl.ds(off, GCH)], sem_w.at[p])
      return carry

    lax.fori_loop(0, GITERS, body, 0)
    for p in range(2):
      off2 = base + (GITERS - 2 + p) * GCH
      pltpu.make_async_copy(rows_v.at[p], out_hbm.at[pl.ds(off2, GCH)],
                            sem_w.at[p]).wait()

  return k(feats, edge_index)


BE = 6400
NB = E // BE


PACK = 128 // C_IN
BR = BE // PACK


def _gemm_tc(gathered4, ek3, wblk):
  def body(ek_ref, x_ref, w_ref, o_ref):
    ek = ek_ref[0, 0, :]
    kmin = jnp.min(ek)
    kmax = jnp.max(ek)

    @pl.when(kmin == kmax)
    def _():
      wk = w_ref[pl.ds(kmin, 1), :, :][0]
      o_ref[...] = jnp.dot(x_ref[...], wk, preferred_element_type=jnp.float32)

    @pl.when(kmin != kmax)
    def _():
      x = x_ref[...]
      eidx = (PACK * lax.broadcasted_iota(jnp.int32, (BR, 128), 0)
              + lax.broadcasted_iota(jnp.int32, (BR, 128), 1) // C_IN)
      o_ref[...] = jnp.zeros_like(o_ref)
      for k in range(KVOL):
        @pl.when(jnp.logical_and(kmin <= k, k <= kmax))
        def _():
          lo = jnp.sum((ek < k).astype(jnp.int32))
          hi = jnp.sum((ek <= k).astype(jnp.int32))
          xm = jnp.where((eidx >= lo) & (eidx < hi), x, 0.0)
          o_ref[...] += jnp.dot(xm, w_ref[k],
                                preferred_element_type=jnp.float32)

  return pl.pallas_call(
      body,
      grid=(ek3.shape[0],),
      in_specs=[
          pl.BlockSpec((1, 1, BE), lambda i: (i, 0, 0)),
          pl.BlockSpec((BR, 128), lambda i: (i, 0)),
          pl.BlockSpec((KVOL, 128, 128), lambda i: (0, 0, 0)),
      ],
      out_specs=pl.BlockSpec((BR, 128), lambda i: (i, 0)),
          out_shape=jax.ShapeDtypeStruct((ek3.shape[0] * BR, 128), jnp.float32),
  )(ek3, gathered4, wblk)


SCH = 400
EPH = EH // NS
SITERS = EPH // SCH
NPT = N // NS
CH = C_OUT // NC
_NCHUNKS = tuple((i * 400, 400) for i in range(15)) + ((6000, 250),)


def _scatter_sc(msgs, edge_index, bias, e0, prev):
  mesh = plsc.VectorSubcoreMesh(core_axis_name="c", subcore_axis_name="s")
  first = prev is None
  in_args = (msgs, edge_index, bias) if first else (msgs, edge_index, bias, prev)

  @functools.partial(
      pl.kernel,
      mesh=mesh,
      compiler_params=pltpu.CompilerParams(use_tc_tiling_on_sc=False),
      out_type=jax.ShapeDtypeStruct((N, C_OUT), jnp.float32),
      scratch_types=[
          pltpu.VMEM((2, SCH), jnp.int32),
          pltpu.VMEM((2, SCH, CH), jnp.float32),
          pltpu.VMEM((L,), jnp.float32),
          pltpu.VMEM_SHARED((N, CH), jnp.float32),
          pltpu.SemaphoreType.DMA((2,)),
          pltpu.SemaphoreType.DMA((2,)),
      ],
  )
  def k(*refs):
    if first:
      msgs_hbm, ei_hbm, bias_hbm, out_hbm = refs[:4]
      prev_hbm = None
    else:
      msgs_hbm, ei_hbm, bias_hbm, prev_hbm, out_hbm = refs[:5]
    idx_v, msg_v, bias_v, acc_sh, sem_i, sem_m = refs[-6:]
    c = lax.axis_index("c")
    s = lax.axis_index("s")
    coff = c * CH
    nb = s * NPT
    buf_v = msg_v.at[0]

    if first:
      def zero_row(i, carry):
        buf_v[i, :] = jnp.zeros((L,), jnp.float32)
        return carry

      lax.fori_loop(0, SCH, zero_row, 0)
      for off, n in _NCHUNKS:
        pltpu.sync_copy(buf_v.at[pl.ds(0, n)], acc_sh.at[pl.ds(nb + off, n)])
    else:
      pltpu.sync_copy(
          prev_hbm.at[pl.ds(nb, NPT), pl.ds(coff, CH)],
          acc_sh.at[pl.ds(nb, NPT)])
    plsc.subcore_barrier()

    ebase = e0 + s * EPH
    lbase = s * EPH

    def _issue(i, p):
      goff = ebase + i * SCH
      loff = lbase + i * SCH
      pltpu.async_copy(ei_hbm.at[1, pl.ds(goff, SCH)], idx_v.at[p],
                       sem_i.at[p])
      pltpu.async_copy(msgs_hbm.at[pl.ds(loff, SCH), pl.ds(coff, CH)],
                       msg_v.at[p], sem_m.at[p])

    _issue(0, 0)
    _issue(1, 1)

    def body(i, carry):
      p = lax.rem(i, 2)
      goff = ebase + i * SCH
      loff = lbase + i * SCH
      pltpu.make_async_copy(ei_hbm.at[1, pl.ds(goff, SCH)], idx_v.at[p],
                            sem_i.at[p]).wait()
      pltpu.make_async_copy(msgs_hbm.at[pl.ds(loff, SCH), pl.ds(coff, CH)],
                            msg_v.at[p], sem_m.at[p]).wait()
      pltpu.sync_copy(msg_v.at[p], acc_sh.at[idx_v.at[p]], add=True)

      @pl.when(i + 2 < SITERS)
      def _():
        _issue(i + 2, p)

      return carry

    lax.fori_loop(0, SITERS, body, 0)
    plsc.subcore_barrier()

    if first:
      pltpu.sync_copy(acc_sh.at[pl.ds(nb, NPT)],
                      out_hbm.at[pl.ds(nb, NPT), pl.ds(coff, CH)])
    else:
      pltpu.sync_copy(bias_hbm.at[pl.ds(coff, CH)], bias_v)
      bvec = bias_v[...]
      for off, n in _NCHUNKS:
        pltpu.sync_copy(acc_sh.at[pl.ds(nb + off, n)], buf_v.at[pl.ds(0, n)])

        def add_bias(r, carry):
          buf_v[r, :] = buf_v[r, :] + bvec
          return carry

        lax.fori_loop(0, n, add_bias, 0)
        pltpu.sync_copy(buf_v.at[pl.ds(0, n)],
                        out_hbm.at[pl.ds(nb + off, n), pl.ds(coff, CH)])

  return k(*in_args)


def kernel(feats, edge_index, edge_kernel, weight, bias):
  w_t = jnp.transpose(weight, (0, 2, 1))
  wblk = jnp.einsum("ab,kio->kaibo", jnp.eye(PACK, dtype=weight.dtype),
                    w_t).reshape(KVOL, 128, 128)
  g1 = _gather_sc(feats, edge_index, 0)
  m1 = _gemm_tc(g1.reshape(EH // PACK, 128),
                edge_kernel[:EH].reshape(EH // BE, 1, BE), wblk)
  g2 = _gather_sc(feats, edge_index, EH)
  m2 = _gemm_tc(g2.reshape(EH // PACK, 128),
                edge_kernel[EH:].reshape(EH // BE, 1, BE), wblk)
  part = _scatter_sc(m1.reshape(EH, C_OUT), edge_index, bias, 0, None)
  return _scatter_sc(m2.reshape(EH, C_OUT), edge_index, bias, EH, part)


# speedup vs baseline: 1.0182x; 1.0182x over previous
import functools

import jax
import jax.numpy as jnp
from jax import lax
from jax.experimental import pallas as pl
from jax.experimental.pallas import tpu as pltpu
from jax.experimental.pallas import tpu_sc as plsc

E = 1_600_000
N = 100_000
C_IN = 32
C_OUT = 32
KVOL = 27

_info = plsc.get_sparse_core_info()
NC = _info.num_cores
NS = _info.num_subcores
L = _info.num_lanes
NW = NC * NS

EH = E // 2
GCH = 1000
EPW = EH // NW
GITERS = EPW // GCH


def _gather_sc(feats, edge_index, e0):
  mesh = plsc.VectorSubcoreMesh(core_axis_name="c", subcore_axis_name="s")

  @functools.partial(
      pl.kernel,
      mesh=mesh,
      compiler_params=pltpu.CompilerParams(use_tc_tiling_on_sc=False),
      out_type=jax.ShapeDtypeStruct((EH, C_IN), jnp.float32),
      scratch_types=[
          pltpu.VMEM((EPW,), jnp.int32),
          pltpu.VMEM((2, GCH, C_IN), jnp.float32),
          pltpu.SemaphoreType.DMA,
          pltpu.SemaphoreType.DMA((2,)),
      ],
  )
  def k(feats_hbm, ei_hbm, out_hbm, idx_v, rows_v, sem_g, sem_w):
    wid = lax.axis_index("s") * NC + lax.axis_index("c")
    base = wid * EPW
    pltpu.sync_copy(ei_hbm.at[0, pl.ds(e0 + base, EPW)], idx_v)

    def body(i, carry):
      p = lax.rem(i, 2)
      off = base + i * GCH

      @pl.when(i >= 2)
      def _():
        off2 = base + (i - 2) * GCH
        pltpu.make_async_copy(rows_v.at[p], out_hbm.at[pl.ds(off2, GCH)],
                              sem_w.at[p]).wait()

      pltpu.async_copy(feats_hbm.at[idx_v.at[pl.ds(i * GCH, GCH)]],
                       rows_v.at[p], sem_g).wait()
      pltpu.async_copy(rows_v.at[p], out_hbm.at[pl.ds(off, GCH)], sem_w.at[p])
      return carry

    lax.fori_loop(0, GITERS, body, 0)
    for p in range(2):
      off2 = base + (GITERS - 2 + p) * GCH
      pltpu.make_async_copy(rows_v.at[p], out_hbm.at[pl.ds(off2, GCH)],
                            sem_w.at[p]).wait()

  return k(feats, edge_index)


BE = 6400
NB = E // BE


PACK = 128 // C_IN
BR = BE // PACK


def _gemm_tc(gathered4, ek3, wblk):
  def body(ek_ref, x_ref, w_ref, o_ref):
    ek = ek_ref[0, 0, :]
    kmin = jnp.min(ek)
    kmax = jnp.max(ek)

    @pl.when(kmin == kmax)
    def _():
      wk = w_ref[pl.ds(kmin, 1), :, :][0]
      o_ref[...] = jnp.dot(x_ref[...], wk, preferred_element_type=jnp.float32)

    @pl.when(kmin != kmax)
    def _():
      x = x_ref[...]
      eidx = (PACK * lax.broadcasted_iota(jnp.int32, (BR, 128), 0)
              + lax.broadcasted_iota(jnp.int32, (BR, 128), 1) // C_IN)
      o_ref[...] = jnp.zeros_like(o_ref)
      for k in range(KVOL):
        @pl.when(jnp.logical_and(kmin <= k, k <= kmax))
        def _():
          lo = jnp.sum((ek < k).astype(jnp.int32))
          hi = jnp.sum((ek <= k).astype(jnp.int32))
          xm = jnp.where((eidx >= lo) & (eidx < hi), x, 0.0)
          o_ref[...] += jnp.dot(xm, w_ref[k],
                                preferred_element_type=jnp.float32)

  return pl.pallas_call(
      body,
      grid=(ek3.shape[0],),
      in_specs=[
          pl.BlockSpec((1, 1, BE), lambda i: (i, 0, 0)),
          pl.BlockSpec((BR, 128), lambda i: (i, 0)),
          pl.BlockSpec((KVOL, 128, 128), lambda i: (0, 0, 0)),
      ],
      out_specs=pl.BlockSpec((BR, 128), lambda i: (i, 0)),
          out_shape=jax.ShapeDtypeStruct((ek3.shape[0] * BR, 128), jnp.float32),
  )(ek3, gathered4, wblk)


SCH = 800
EPT = E // NS
SITERS = EPT // SCH
NPT = N // NS
CH = C_OUT // NC
_NCHUNKS = tuple((i * 800, 800) for i in range(7)) + ((5600, 650),)


def _scatter_sc(msgs1, msgs2, edge_index, bias):
  mesh = plsc.VectorSubcoreMesh(core_axis_name="c", subcore_axis_name="s")

  @functools.partial(
      pl.kernel,
      mesh=mesh,
      compiler_params=pltpu.CompilerParams(use_tc_tiling_on_sc=False),
      out_type=jax.ShapeDtypeStruct((N, C_OUT), jnp.float32),
      scratch_types=[
          pltpu.VMEM((2, SCH), jnp.int32),
          pltpu.VMEM((2, SCH, CH), jnp.float32),
          pltpu.VMEM((L,), jnp.float32),
          pltpu.VMEM_SHARED((N, CH), jnp.float32),
          pltpu.SemaphoreType.DMA((2,)),
          pltpu.SemaphoreType.DMA((2,)),
      ],
  )
  def k(m1_hbm, m2_hbm, ei_hbm, bias_hbm, out_hbm,
        idx_v, msg_v, bias_v, acc_sh, sem_i, sem_m):
    c = lax.axis_index("c")
    s = lax.axis_index("s")
    coff = c * CH
    nb = s * NPT
    buf_v = msg_v.at[0]

    def zero_row(i, carry):
      buf_v[i, :] = jnp.zeros((L,), jnp.float32)
      return carry

    lax.fori_loop(0, SCH, zero_row, 0)
    for off, n in _NCHUNKS:
      pltpu.sync_copy(buf_v.at[pl.ds(0, n)], acc_sh.at[pl.ds(nb + off, n)])
    plsc.subcore_barrier()

    def run(msgs_hbm, ebase, lbase):
      def _issue(i, p):
        goff = ebase + i * SCH
        loff = lbase + i * SCH
        pltpu.async_copy(ei_hbm.at[1, pl.ds(goff, SCH)], idx_v.at[p],
                         sem_i.at[p])
        pltpu.async_copy(msgs_hbm.at[pl.ds(loff, SCH), pl.ds(coff, CH)],
                         msg_v.at[p], sem_m.at[p])

      _issue(0, 0)
      _issue(1, 1)

      def body(i, carry):
        p = lax.rem(i, 2)
        goff = ebase + i * SCH
        loff = lbase + i * SCH
        pltpu.make_async_copy(ei_hbm.at[1, pl.ds(goff, SCH)], idx_v.at[p],
                              sem_i.at[p]).wait()
        pltpu.make_async_copy(msgs_hbm.at[pl.ds(loff, SCH), pl.ds(coff, CH)],
                              msg_v.at[p], sem_m.at[p]).wait()
        pltpu.sync_copy(msg_v.at[p], acc_sh.at[idx_v.at[p]], add=True)

        @pl.when(i + 2 < SITERS)
        def _():
          _issue(i + 2, p)

        return carry

      lax.fori_loop(0, SITERS, body, 0)

    @pl.when(s < NS // 2)
    def _():
      run(m1_hbm, s * EPT, s * EPT)

    @pl.when(s >= NS // 2)
    def _():
      run(m2_hbm, s * EPT, s * EPT - EH)

    plsc.subcore_barrier()

    pltpu.sync_copy(bias_hbm.at[pl.ds(coff, CH)], bias_v)
    bvec = bias_v[...]
    for off, n in _NCHUNKS:
      pltpu.sync_copy(acc_sh.at[pl.ds(nb + off, n)], buf_v.at[pl.ds(0, n)])

      def add_bias(r, carry):
        buf_v[r, :] = buf_v[r, :] + bvec
        return carry

      lax.fori_loop(0, n, add_bias, 0)
      pltpu.sync_copy(buf_v.at[pl.ds(0, n)],
                      out_hbm.at[pl.ds(nb + off, n), pl.ds(coff, CH)])

  return k(msgs1, msgs2, edge_index, bias)


def kernel(feats, edge_index, edge_kernel, weight, bias):
  w_t = jnp.transpose(weight, (0, 2, 1))
  wblk = jnp.einsum("ab,kio->kaibo", jnp.eye(PACK, dtype=weight.dtype),
                    w_t).reshape(KVOL, 128, 128)
  g1 = _gather_sc(feats, edge_index, 0)
  m1 = _gemm_tc(g1.reshape(EH // PACK, 128),
                edge_kernel[:EH].reshape(EH // BE, 1, BE), wblk)
  g2 = _gather_sc(feats, edge_index, EH)
  m2 = _gemm_tc(g2.reshape(EH // PACK, 128),
                edge_kernel[EH:].reshape(EH // BE, 1, BE), wblk)
  return _scatter_sc(m1.reshape(EH, C_OUT), m2.reshape(EH, C_OUT),
                     edge_index, bias)
